# baseline (speedup 1.0000x reference)
import functools

import jax
import jax.numpy as jnp
from jax import lax
from jax.experimental import pallas as pl
from jax.experimental.pallas import tpu as pltpu
from jax.experimental.pallas import tpu_sc as plsc

N_ROWS = 1000000
B_SIZE = 16384

NC = 2
NS = 16
NW = NC * NS
BPW = B_SIZE // NW
IDX_ROWS = BPW // 128

SLICE = 31744
N_CHUNKS = 2
CHUNK = SLICE // N_CHUNKS
UNROLL = 4
N_ITER = CHUNK // (16 * UNROLL)


@functools.cache
def _sc_gather_and_max():
    mesh = plsc.VectorSubcoreMesh(
        core_axis_name="c", subcore_axis_name="s", num_cores=NC, num_subcores=NS
    )

    @functools.partial(
        pl.kernel,
        out_type=(
            jax.ShapeDtypeStruct((B_SIZE // 128, 128), jnp.float32),
            jax.ShapeDtypeStruct((NW, 16), jnp.float32),
        ),
        mesh=mesh,
        scratch_types=(
            pltpu.VMEM((IDX_ROWS, 128), jnp.int32),
            pltpu.VMEM((IDX_ROWS, 128), jnp.float32),
            pltpu.VMEM((SLICE,), jnp.float32),
            pltpu.VMEM((16,), jnp.float32),
            pltpu.SemaphoreType.DMA,
            pltpu.SemaphoreType.DMA,
        ),
    )
    def k(item_hbm, tbl_hbm, outg_hbm, outp_hbm, idx_v, rows_v, tbl_v, pm_v,
          sem_g, sem_t):
        wid = lax.axis_index("s") * NC + lax.axis_index("c")
        tbl_lin = tbl_hbm.at[0]

        pltpu.sync_copy(item_hbm.at[pl.ds(wid * IDX_ROWS, IDX_ROWS)], idx_v)
        gathers = [
            pltpu.async_copy(tbl_lin.at[idx_v.at[j]], rows_v.at[j], sem_g)
            for j in range(IDX_ROWS)
        ]

        start = jnp.minimum(wid * SLICE, N_ROWS - SLICE)
        chunks = [
            pltpu.async_copy(
                tbl_lin.at[pl.ds(start + c * CHUNK, CHUNK)],
                tbl_v.at[pl.ds(c * CHUNK, CHUNK)],
                sem_t,
            )
            for c in range(N_CHUNKS)
        ]

        accs = tuple(
            jnp.full((16,), -jnp.inf, jnp.float32) for _ in range(UNROLL)
        )
        for c in range(N_CHUNKS):
            chunks[c].wait()

            def body(i, a, _c=c):
                base = _c * CHUNK + i * (UNROLL * 16)
                return tuple(
                    jnp.maximum(a[j], tbl_v[pl.ds(base + j * 16, 16)])
                    for j in range(UNROLL)
                )

            accs = lax.fori_loop(0, N_ITER, body, accs)
        acc = functools.reduce(jnp.maximum, accs)

        pm_v[...] = acc
        pltpu.sync_copy(pm_v, outp_hbm.at[wid])

        for g in gathers:
            g.wait()
        pltpu.sync_copy(rows_v, outg_hbm.at[pl.ds(wid * IDX_ROWS, IDX_ROWS)])

    return k


def _combine_body(g_ref, p_ref, o_ref):
    g = g_ref[...]
    mc = jnp.maximum(jnp.max(p_ref[...]), jnp.max(g) + 1.0)
    o_ref[...] = (g + 1.0) / mc


def kernel(item_cnt, item):
    tbl1x = item_cnt.reshape(1, N_ROWS)
    item2d = item.reshape(B_SIZE // 128, 128)
    gathered2d, partials = _sc_gather_and_max()(item2d, tbl1x)
    out2d = pl.pallas_call(
        _combine_body,
        out_shape=jax.ShapeDtypeStruct((B_SIZE // 128, 128), jnp.float32),
        compiler_params=pltpu.CompilerParams(skip_device_barrier=True),
    )(gathered2d, partials)
    return out2d.reshape(B_SIZE)


# speedup vs baseline: 1.0565x; 1.0565x over previous
import functools

import jax
import jax.numpy as jnp
from jax import lax
from jax.experimental import pallas as pl
from jax.experimental.pallas import tpu as pltpu
from jax.experimental.pallas import tpu_sc as plsc

N_ROWS = 1000000
B_SIZE = 16384

NC = 2
NS = 16
NW = NC * NS
BPW = B_SIZE // NW
IDX_ROWS = BPW // 128

SLICE = 31744
N_CHUNKS = 4
CHUNK = SLICE // N_CHUNKS
UNROLL = 16
N_ITER = CHUNK // (16 * UNROLL)


@functools.cache
def _sc_gather_and_max():
    mesh = plsc.VectorSubcoreMesh(
        core_axis_name="c", subcore_axis_name="s", num_cores=NC, num_subcores=NS
    )

    @functools.partial(
        pl.kernel,
        out_type=(
            jax.ShapeDtypeStruct((B_SIZE // 128, 128), jnp.float32),
            jax.ShapeDtypeStruct((NW, 16), jnp.float32),
        ),
        mesh=mesh,
        scratch_types=(
            pltpu.VMEM((IDX_ROWS, 128), jnp.int32),
            pltpu.VMEM((IDX_ROWS, 128), jnp.float32),
            pltpu.VMEM((SLICE,), jnp.float32),
            pltpu.VMEM((16,), jnp.float32),
            pltpu.SemaphoreType.DMA,
            pltpu.SemaphoreType.DMA,
        ),
    )
    def k(item_hbm, tbl_hbm, outg_hbm, outp_hbm, idx_v, rows_v, tbl_v, pm_v,
          sem_g, sem_t):
        wid = lax.axis_index("s") * NC + lax.axis_index("c")
        tbl_lin = tbl_hbm.at[0]

        start = jnp.minimum(wid * SLICE, N_ROWS - SLICE)
        chunks = [
            pltpu.async_copy(
                tbl_lin.at[pl.ds(start + c * CHUNK, CHUNK)],
                tbl_v.at[pl.ds(c * CHUNK, CHUNK)],
                sem_t,
            )
            for c in range(N_CHUNKS)
        ]
        pltpu.sync_copy(item_hbm.at[pl.ds(wid * IDX_ROWS, IDX_ROWS)], idx_v)
        gathers = [
            pltpu.async_copy(tbl_lin.at[idx_v.at[j]], rows_v.at[j], sem_g)
            for j in range(IDX_ROWS)
        ]

        accs = tuple(
            jnp.full((16,), -jnp.inf, jnp.float32) for _ in range(UNROLL)
        )
        for c in range(N_CHUNKS):
            chunks[c].wait()

            def body(i, a, _c=c):
                base = _c * CHUNK + i * (UNROLL * 16)
                return tuple(
                    jnp.maximum(a[j], tbl_v[pl.ds(base + j * 16, 16)])
                    for j in range(UNROLL)
                )

            accs = lax.fori_loop(0, N_ITER, body, accs)
        acc = functools.reduce(jnp.maximum, accs)

        for g in gathers:
            g.wait()
        gacc = jnp.full((16,), -jnp.inf, jnp.float32)
        for j in range(IDX_ROWS):
            for q in range(8):
                gacc = jnp.maximum(gacc, rows_v[j, pl.ds(q * 16, 16)])
        acc = jnp.maximum(acc, gacc + 1.0)

        pm_v[...] = acc
        pltpu.sync_copy(pm_v, outp_hbm.at[wid])
        pltpu.sync_copy(rows_v, outg_hbm.at[pl.ds(wid * IDX_ROWS, IDX_ROWS)])

    return k


def _combine_body(g_ref, p_ref, o_ref):
    mc = jnp.max(p_ref[...])
    o_ref[...] = (g_ref[...] + 1.0) / mc


def kernel(item_cnt, item):
    tbl1x = item_cnt.reshape(1, N_ROWS)
    item2d = item.reshape(B_SIZE // 128, 128)
    gathered2d, partials = _sc_gather_and_max()(item2d, tbl1x)
    out2d = pl.pallas_call(
        _combine_body,
        out_shape=jax.ShapeDtypeStruct((B_SIZE // 128, 128), jnp.float32),
    )(gathered2d, partials)
    return out2d.reshape(B_SIZE)
